# baseline (speedup 1.0000x reference)
import functools

import jax
import jax.numpy as jnp
from jax import lax
from jax.experimental import pallas as pl
from jax.experimental.pallas import tpu as pltpu
from jax.experimental.pallas import tpu_sc as plsc

B = 4096
C = 1000
N_NODES = 999
S = 2 * N_NODES
S_PAD = 2048
M = 10000

NC = 2
NS = 16
NW = NC * NS
C_PAD = 1024
ROWS = C_PAD // NW
SLAB = ROWS * S_PAD
M_PAD = M + 16


def _sc_body(ci_hbm, si_hbm, out_hbm, slab, ci_v, si_v):
  core = lax.axis_index("c")
  tid = lax.axis_index("s")
  wid = core * NS + tid

  pltpu.sync_copy(ci_hbm.at[pl.ds(0, M)], ci_v.at[pl.ds(0, M)])
  pltpu.sync_copy(si_hbm.at[pl.ds(0, M)], si_v.at[pl.ds(0, M)])

  zero = jnp.zeros((16,), jnp.float32)

  def zfill(j, _):
    slab[pl.ds(j * 16, 16)] = zero
    return _

  lax.fori_loop(0, SLAB // 16, zfill, 0, unroll=8)

  row_base = wid * ROWS
  lane = lax.iota(jnp.int32, 16)
  ones = jnp.ones((16,), jnp.float32)
  zkey = jnp.zeros((16,), jnp.int32)

  def acc(i, _):
    ci = ci_v[pl.ds(i * 16, 16)]
    si = si_v[pl.ds(i * 16, 16)]
    local = ci - row_base
    ok = (local >= 0) & (local < ROWS) & (lane + i * 16 < M)
    key = jnp.where(ok, local * S_PAD + si, zkey)
    plsc.addupdate_scatter(slab, [key], ones, mask=ok)
    return _

  lax.fori_loop(0, M_PAD // 16, acc, 0, unroll=4)

  pltpu.sync_copy(slab, out_hbm.at[pl.ds(wid * SLAB, SLAB)])


def _build_a(class_index, segment_ids):
  mesh = plsc.VectorSubcoreMesh(core_axis_name="c", subcore_axis_name="s")
  fn = pl.kernel(
      _sc_body,
      out_type=jax.ShapeDtypeStruct((C_PAD * S_PAD,), jnp.float32),
      mesh=mesh,
      compiler_params=pltpu.CompilerParams(needs_layout_passes=False),
      scratch_types=[
          pltpu.VMEM((SLAB,), jnp.float32),
          pltpu.VMEM((M_PAD,), jnp.int32),
          pltpu.VMEM((M_PAD,), jnp.int32),
      ],
  )
  return fn(class_index, segment_ids).reshape(C_PAD, S_PAD)


BM = 256
CHUNK = 512


def _tc_body(x_ref, a_ref, logits_ref, probs_ref, preds_ref, inv_ref):
  i = pl.program_id(0)

  @pl.when(i == 0)
  def _():
    a = a_ref[...]
    cnt = jnp.maximum(jnp.sum(a, axis=0, keepdims=True), 1.0)
    inv_ref[...] = 1.0 / cnt

  x = x_ref[...]
  a = a_ref[...]
  sums = jnp.dot(x, a, preferred_element_type=jnp.float32,
                 precision=lax.Precision.HIGHEST)

  for c0 in range(0, S_PAD, CHUNK):
    scaled = sums[:, c0:c0 + CHUNK] * inv_ref[:, c0:c0 + CHUNK]

    shl = pltpu.roll(scaled, CHUNK - 1, 1)
    shr = pltpu.roll(scaled, 1, 1)
    lane = lax.broadcasted_iota(jnp.int32, (BM, CHUNK), 1)
    partner = jnp.where((lane % 2) == 0, shl, shr)

    mx = jnp.maximum(scaled, partner)
    e = jnp.exp(scaled - mx)
    ep = jnp.exp(partner - mx)
    probs = e / (e + ep)

    w = min(CHUNK, S - c0)
    logits_ref[:, c0:c0 + w] = scaled[:, :w]
    probs_ref[:, c0:c0 + w] = probs[:, :w]

    g = jnp.where(partner > scaled, 1, 0)
    gc = g.reshape(BM, CHUNK // 2, 2)[:, :, 0]
    p0 = c0 // 2
    wp = min(CHUNK // 2, N_NODES - p0)
    preds_ref[:, p0:p0 + wp] = gc[:, :wp]


def _decide(outputs, a_mat):
  grid = (B // BM,)
  return pl.pallas_call(
      _tc_body,
      grid=grid,
      in_specs=[
          pl.BlockSpec((BM, C), lambda i: (i, 0)),
          pl.BlockSpec((C, S_PAD), lambda i: (0, 0)),
      ],
      out_specs=[
          pl.BlockSpec((BM, S), lambda i: (i, 0)),
          pl.BlockSpec((BM, S), lambda i: (i, 0)),
          pl.BlockSpec((BM, N_NODES), lambda i: (i, 0)),
      ],
      out_shape=[
          jax.ShapeDtypeStruct((B, S), jnp.float32),
          jax.ShapeDtypeStruct((B, S), jnp.float32),
          jax.ShapeDtypeStruct((B, N_NODES), jnp.int32),
      ],
      scratch_shapes=[pltpu.VMEM((1, S_PAD), jnp.float32)],
      compiler_params=pltpu.CompilerParams(
          vmem_limit_bytes=56 * 1024 * 1024),
  )(outputs, a_mat)


@jax.jit
def kernel(outputs, class_index, segment_ids):
  a_mat = _build_a(class_index, segment_ids)
  logits_flat, probs_flat, preds = _decide(outputs, a_mat)
  node_logits = logits_flat.reshape(B, N_NODES, 2)
  probs = probs_flat.reshape(B, N_NODES, 2)
  return node_logits, preds, probs


# speedup vs baseline: 3.4983x; 3.4983x over previous
import jax
import jax.numpy as jnp
from jax import lax
from jax.experimental import pallas as pl
from jax.experimental.pallas import tpu as pltpu
from jax.experimental.pallas import tpu_sc as plsc

B = 4096
C = 1000
N_NODES = 999
S = 2 * N_NODES
NP = 1024
M = 10000

NC = 2
NS = 16
NW = NC * NS
C_PAD = 1024
ROWS = C_PAD // NW
SLAB = ROWS * NP
M_PAD = M + 16


def _sc_body(ci_hbm, si_hbm, out0_hbm, out1_hbm, slab0, slab1, ci_v, si_v):
  core = lax.axis_index("c")
  tid = lax.axis_index("s")
  wid = core * NS + tid

  pltpu.sync_copy(ci_hbm.at[pl.ds(0, M)], ci_v.at[pl.ds(0, M)])
  pltpu.sync_copy(si_hbm.at[pl.ds(0, M)], si_v.at[pl.ds(0, M)])

  zero = jnp.zeros((16,), jnp.float32)

  def zfill(j, _):
    slab0[pl.ds(j * 16, 16)] = zero
    slab1[pl.ds(j * 16, 16)] = zero
    return _

  lax.fori_loop(0, SLAB // 16, zfill, 0, unroll=8)

  row_base = wid * ROWS
  lane = lax.iota(jnp.int32, 16)
  ones = jnp.ones((16,), jnp.float32)
  zkey = jnp.zeros((16,), jnp.int32)

  def acc(i, _):
    ci = ci_v[pl.ds(i * 16, 16)]
    si = si_v[pl.ds(i * 16, 16)]
    local = ci - row_base
    ok = (local >= 0) & (local < ROWS) & (lane + i * 16 < M)
    odd = (si & 1) == 1
    key = local * NP + (si >> 1)
    key0 = jnp.where(ok & ~odd, key, zkey)
    key1 = jnp.where(ok & odd, key, zkey)
    plsc.addupdate_scatter(slab0, [key0], ones, mask=ok & ~odd)
    plsc.addupdate_scatter(slab1, [key1], ones, mask=ok & odd)
    return _

  lax.fori_loop(0, M_PAD // 16, acc, 0, unroll=4)

  pltpu.sync_copy(slab0, out0_hbm.at[pl.ds(wid * SLAB, SLAB)])
  pltpu.sync_copy(slab1, out1_hbm.at[pl.ds(wid * SLAB, SLAB)])


def _build_a(class_index, segment_ids):
  mesh = plsc.VectorSubcoreMesh(core_axis_name="c", subcore_axis_name="s")
  fn = pl.kernel(
      _sc_body,
      out_type=[
          jax.ShapeDtypeStruct((C_PAD * NP,), jnp.float32),
          jax.ShapeDtypeStruct((C_PAD * NP,), jnp.float32),
      ],
      mesh=mesh,
      compiler_params=pltpu.CompilerParams(needs_layout_passes=False),
      scratch_types=[
          pltpu.VMEM((SLAB,), jnp.float32),
          pltpu.VMEM((SLAB,), jnp.float32),
          pltpu.VMEM((M_PAD,), jnp.int32),
          pltpu.VMEM((M_PAD,), jnp.int32),
      ],
  )
  a0, a1 = fn(class_index, segment_ids)
  return a0.reshape(C_PAD, NP), a1.reshape(C_PAD, NP)


BM = 256


def _hi_lo(v):
  hi = v.astype(jnp.bfloat16)
  lo = (v - hi.astype(jnp.float32)).astype(jnp.bfloat16)
  return hi, lo


def _tc_body(x_ref, a0_ref, a1_ref, e0_ref, e1_ref,
             logits_ref, probs_ref, preds_ref, inv0_ref, inv1_ref):
  i = pl.program_id(0)

  @pl.when(i == 0)
  def _():
    inv0_ref[...] = 1.0 / jnp.maximum(
        jnp.sum(a0_ref[...], axis=0, keepdims=True), 1.0)
    inv1_ref[...] = 1.0 / jnp.maximum(
        jnp.sum(a1_ref[...], axis=0, keepdims=True), 1.0)

  x = x_ref[...]
  hp = lax.Precision.HIGHEST
  l0 = jnp.dot(x, a0_ref[...], preferred_element_type=jnp.float32,
               precision=hp) * inv0_ref[...]
  l1 = jnp.dot(x, a1_ref[...], preferred_element_type=jnp.float32,
               precision=hp) * inv1_ref[...]

  preds_ref[...] = jnp.where(l1 > l0, 1, 0)[:, :N_NODES]

  m = jnp.maximum(l0, l1)
  e0 = jnp.exp(l0 - m)
  e1 = jnp.exp(l1 - m)
  d = 1.0 / (e0 + e1)
  p0 = e0 * d
  p1 = e1 * d

  e0m = e0_ref[...]
  e1m = e1_ref[...]

  def interleave(v0, v1):
    h0, w0 = _hi_lo(v0)
    h1, w1 = _hi_lo(v1)
    acc = jnp.dot(h0, e0m, preferred_element_type=jnp.float32)
    acc += jnp.dot(w0, e0m, preferred_element_type=jnp.float32)
    acc += jnp.dot(h1, e1m, preferred_element_type=jnp.float32)
    acc += jnp.dot(w1, e1m, preferred_element_type=jnp.float32)
    return acc

  logits_ref[...] = interleave(l0, l1)[:, :S]
  probs_ref[...] = interleave(p0, p1)[:, :S]


def _decide(outputs, a0, a1, e0m, e1m):
  grid = (B // BM,)
  return pl.pallas_call(
      _tc_body,
      grid=grid,
      in_specs=[
          pl.BlockSpec((BM, C), lambda i: (i, 0)),
          pl.BlockSpec((C, NP), lambda i: (0, 0)),
          pl.BlockSpec((C, NP), lambda i: (0, 0)),
          pl.BlockSpec((NP, 2 * NP), lambda i: (0, 0)),
          pl.BlockSpec((NP, 2 * NP), lambda i: (0, 0)),
      ],
      out_specs=[
          pl.BlockSpec((BM, S), lambda i: (i, 0)),
          pl.BlockSpec((BM, S), lambda i: (i, 0)),
          pl.BlockSpec((BM, N_NODES), lambda i: (i, 0)),
      ],
      out_shape=[
          jax.ShapeDtypeStruct((B, S), jnp.float32),
          jax.ShapeDtypeStruct((B, S), jnp.float32),
          jax.ShapeDtypeStruct((B, N_NODES), jnp.int32),
      ],
      scratch_shapes=[
          pltpu.VMEM((1, NP), jnp.float32),
          pltpu.VMEM((1, NP), jnp.float32),
      ],
      compiler_params=pltpu.CompilerParams(
          vmem_limit_bytes=56 * 1024 * 1024),
  )(outputs, a0, a1, e0m, e1m)


def _selection_mats():
  col = lax.broadcasted_iota(jnp.int32, (NP, 2 * NP), 1)
  row = lax.broadcasted_iota(jnp.int32, (NP, 2 * NP), 0)
  e0m = (col == 2 * row).astype(jnp.bfloat16)
  e1m = (col == 2 * row + 1).astype(jnp.bfloat16)
  return e0m, e1m


@jax.jit
def kernel(outputs, class_index, segment_ids):
  a0, a1 = _build_a(class_index, segment_ids)
  e0m, e1m = _selection_mats()
  logits_flat, probs_flat, preds = _decide(outputs, a0, a1, e0m, e1m)
  node_logits = logits_flat.reshape(B, N_NODES, 2)
  probs = probs_flat.reshape(B, N_NODES, 2)
  return node_logits, preds, probs


# speedup vs baseline: 4.1414x; 1.1838x over previous
import jax
import jax.numpy as jnp
from jax import lax
from jax.experimental import pallas as pl
from jax.experimental.pallas import tpu as pltpu
from jax.experimental.pallas import tpu_sc as plsc

B = 4096
C = 1000
N_NODES = 999
S = 2 * N_NODES
NP = 1024
M = 10000

NC = 2
NS = 16
NW = NC * NS
C_PAD = 1024
ROWS = C_PAD // NW
SLAB = ROWS * NP
M_PAD = M + 16


def _sc_body(ci_hbm, si_hbm, out0_hbm, out1_hbm, slab0, slab1, ci_v, si_v):
  core = lax.axis_index("c")
  tid = lax.axis_index("s")
  wid = core * NS + tid

  pltpu.sync_copy(ci_hbm.at[pl.ds(0, M)], ci_v.at[pl.ds(0, M)])
  pltpu.sync_copy(si_hbm.at[pl.ds(0, M)], si_v.at[pl.ds(0, M)])

  zero = jnp.zeros((16,), jnp.float32)

  def zfill(j, _):
    slab0[pl.ds(j * 16, 16)] = zero
    slab1[pl.ds(j * 16, 16)] = zero
    return _

  lax.fori_loop(0, SLAB // 16, zfill, 0, unroll=8)

  row_base = wid * ROWS
  lane = lax.iota(jnp.int32, 16)
  ones = jnp.ones((16,), jnp.float32)
  zkey = jnp.zeros((16,), jnp.int32)

  def acc(i, _):
    ci = ci_v[pl.ds(i * 16, 16)]
    si = si_v[pl.ds(i * 16, 16)]
    local = ci - row_base
    ok = (local >= 0) & (local < ROWS) & (lane + i * 16 < M)
    odd = (si & 1) == 1
    key = local * NP + (si >> 1)
    key0 = jnp.where(ok & ~odd, key, zkey)
    key1 = jnp.where(ok & odd, key, zkey)
    plsc.addupdate_scatter(slab0, [key0], ones, mask=ok & ~odd)
    plsc.addupdate_scatter(slab1, [key1], ones, mask=ok & odd)
    return _

  lax.fori_loop(0, M_PAD // 16, acc, 0, unroll=4)

  pltpu.sync_copy(slab0, out0_hbm.at[pl.ds(wid * SLAB, SLAB)])
  pltpu.sync_copy(slab1, out1_hbm.at[pl.ds(wid * SLAB, SLAB)])


def _build_a(class_index, segment_ids):
  mesh = plsc.VectorSubcoreMesh(core_axis_name="c", subcore_axis_name="s")
  fn = pl.kernel(
      _sc_body,
      out_type=[
          jax.ShapeDtypeStruct((C_PAD * NP,), jnp.float32),
          jax.ShapeDtypeStruct((C_PAD * NP,), jnp.float32),
      ],
      mesh=mesh,
      compiler_params=pltpu.CompilerParams(needs_layout_passes=False),
      scratch_types=[
          pltpu.VMEM((SLAB,), jnp.float32),
          pltpu.VMEM((SLAB,), jnp.float32),
          pltpu.VMEM((M_PAD,), jnp.int32),
          pltpu.VMEM((M_PAD,), jnp.int32),
      ],
  )
  a0, a1 = fn(class_index, segment_ids)
  return a0.reshape(C_PAD, NP), a1.reshape(C_PAD, NP)


BM = 256


def _hi_lo(v):
  hi = v.astype(jnp.bfloat16)
  lo = (v - hi.astype(jnp.float32)).astype(jnp.bfloat16)
  return hi, lo


def _tc_body(x_ref, a0_ref, a1_ref, e0_ref, e1_ref,
             logits_ref, probs_ref, preds_ref, inv0_ref, inv1_ref):
  i = pl.program_id(0)

  @pl.when(i == 0)
  def _():
    inv0_ref[...] = 1.0 / jnp.maximum(
        jnp.sum(a0_ref[...], axis=0, keepdims=True,
                dtype=jnp.float32), 1.0)
    inv1_ref[...] = 1.0 / jnp.maximum(
        jnp.sum(a1_ref[...], axis=0, keepdims=True,
                dtype=jnp.float32), 1.0)

  xh, xw = _hi_lo(x_ref[...])
  a0 = a0_ref[...]
  a1 = a1_ref[...]

  def mm(h, w, a):
    r = jnp.dot(h, a, preferred_element_type=jnp.float32)
    return r + jnp.dot(w, a, preferred_element_type=jnp.float32)

  l0 = mm(xh, xw, a0) * inv0_ref[...]
  l1 = mm(xh, xw, a1) * inv1_ref[...]

  preds_ref[...] = jnp.where(l1 > l0, 1, 0)[:, :N_NODES]

  m = jnp.maximum(l0, l1)
  e0 = jnp.exp(l0 - m)
  e1 = jnp.exp(l1 - m)
  d = 1.0 / (e0 + e1)
  p0 = e0 * d
  p1 = e1 * d

  e0m = e0_ref[...]
  e1m = e1_ref[...]

  def interleave(v0, v1):
    h0, w0 = _hi_lo(v0)
    h1, w1 = _hi_lo(v1)
    acc = jnp.dot(h0, e0m, preferred_element_type=jnp.float32)
    acc += jnp.dot(w0, e0m, preferred_element_type=jnp.float32)
    acc += jnp.dot(h1, e1m, preferred_element_type=jnp.float32)
    acc += jnp.dot(w1, e1m, preferred_element_type=jnp.float32)
    return acc

  logits_ref[...] = interleave(l0, l1)[:, :S]
  probs_ref[...] = interleave(p0, p1)[:, :S]


def _decide(outputs, a0, a1, e0m, e1m):
  grid = (B // BM,)
  return pl.pallas_call(
      _tc_body,
      grid=grid,
      in_specs=[
          pl.BlockSpec((BM, C), lambda i: (i, 0)),
          pl.BlockSpec((C, NP), lambda i: (0, 0)),
          pl.BlockSpec((C, NP), lambda i: (0, 0)),
          pl.BlockSpec((NP, 2 * NP), lambda i: (0, 0)),
          pl.BlockSpec((NP, 2 * NP), lambda i: (0, 0)),
      ],
      out_specs=[
          pl.BlockSpec((BM, S), lambda i: (i, 0)),
          pl.BlockSpec((BM, S), lambda i: (i, 0)),
          pl.BlockSpec((BM, N_NODES), lambda i: (i, 0)),
      ],
      out_shape=[
          jax.ShapeDtypeStruct((B, S), jnp.float32),
          jax.ShapeDtypeStruct((B, S), jnp.float32),
          jax.ShapeDtypeStruct((B, N_NODES), jnp.int32),
      ],
      scratch_shapes=[
          pltpu.VMEM((1, NP), jnp.float32),
          pltpu.VMEM((1, NP), jnp.float32),
      ],
      compiler_params=pltpu.CompilerParams(
          vmem_limit_bytes=56 * 1024 * 1024),
  )(outputs, a0, a1, e0m, e1m)


def _selection_mats():
  col = lax.broadcasted_iota(jnp.int32, (NP, 2 * NP), 1)
  row = lax.broadcasted_iota(jnp.int32, (NP, 2 * NP), 0)
  e0m = (col == 2 * row).astype(jnp.bfloat16)
  e1m = (col == 2 * row + 1).astype(jnp.bfloat16)
  return e0m, e1m


@jax.jit
def kernel(outputs, class_index, segment_ids):
  a0, a1 = _build_a(class_index, segment_ids)
  a0 = a0.astype(jnp.bfloat16)
  a1 = a1.astype(jnp.bfloat16)
  e0m, e1m = _selection_mats()
  logits_flat, probs_flat, preds = _decide(outputs, a0, a1, e0m, e1m)
  node_logits = logits_flat.reshape(B, N_NODES, 2)
  probs = probs_flat.reshape(B, N_NODES, 2)
  return node_logits, preds, probs
